# baseline (speedup 1.0000x reference)
import functools

import jax
import jax.numpy as jnp
from jax import lax
from jax.experimental import pallas as pl
from jax.experimental.pallas import tpu as pltpu
from jax.experimental.pallas import tpu_sc as plsc

NUM_CODEBOOKS = 8
CODEBOOK_SIZE = 512
CODE_DIM = 64
BATCH = 1024
_PAIRS = NUM_CODEBOOKS // 2
_NW = 32
_EIGHTHS = _NW // _PAIRS
_BE = BATCH // _EIGHTHS


def _vq_tc_kernel(z_ref, cb_ref, idx_ref, gidx_ref, counts_ref, scal_ref,
                  commit_smem):
    j = pl.program_id(0)
    z2 = z_ref[...]

    @pl.when(j == 0)
    def _init():
        counts_ref[...] = jnp.zeros_like(counts_ref)
        commit_smem[0] = 0.0

    kf = jax.lax.broadcasted_iota(
        jnp.int32, (BATCH, CODEBOOK_SIZE), 1).astype(jnp.float32)
    lane_g = jax.lax.broadcasted_iota(jnp.int32, (BATCH, NUM_CODEBOOKS), 1)

    gidx_rows = []
    cnt_total = None
    csum = jnp.sum(z2 * z2)
    for h in range(2):
        zg = z2[:, h * CODE_DIM:(h + 1) * CODE_DIM]
        c = cb_ref[h]
        c_sq = jnp.sum(c * c, axis=1, keepdims=True)
        cross = jax.lax.dot_general(
            zg, c, (((1,), (1,)), ((), ())),
            preferred_element_type=jnp.float32)
        d = jnp.transpose(c_sq) - 2.0 * cross
        dmin = jnp.min(d, axis=1, keepdims=True)
        idx_f = jnp.min(jnp.where(d <= dmin, kf, float(CODEBOOK_SIZE)),
                        axis=1, keepdims=True)
        one_hot = (kf == idx_f).astype(jnp.float32)

        cnt = jnp.sum(one_hot, axis=0, keepdims=True)
        cnt_total = cnt if cnt_total is None else cnt_total + cnt
        csum = csum + jnp.sum(dmin)
        idx_col = idx_f.astype(jnp.int32)
        idx_ref[...] = jnp.where(lane_g == 2 * j + h, idx_col, idx_ref[...])
        gidx_rows.append(
            jnp.transpose(idx_col + (2 * j + h) * CODEBOOK_SIZE))

    gidx_ref[0] = jnp.concatenate(gidx_rows, axis=0)
    counts_ref[...] += cnt_total
    commit_smem[0] += csum

    @pl.when(j == _PAIRS - 1)
    def _finish():
        usage = counts_ref[...] / float(BATCH * NUM_CODEBOOKS)
        ent = -jnp.sum(usage * jnp.log(usage + 1e-10))
        commit = commit_smem[0] / float(BATCH * NUM_CODEBOOKS * CODE_DIM)
        lane = jax.lax.broadcasted_iota(jnp.int32, (1, 128), 1)
        out = jnp.where(lane == 0, commit,
                        jnp.where(lane == 1, ent, jnp.exp(ent)))
        scal_ref[...] = out


_SC_MESH = plsc.VectorSubcoreMesh(core_axis_name="c", subcore_axis_name="s")


_QUARTERS = _NW // NUM_CODEBOOKS
_BQ = BATCH // _QUARTERS


@functools.partial(
    pl.kernel,
    mesh=_SC_MESH,
    out_type=jax.ShapeDtypeStruct((BATCH, NUM_CODEBOOKS * CODE_DIM),
                                  jnp.float32),
    scratch_types=[
        pltpu.VMEM((_BQ,), jnp.int32),
        pltpu.VMEM((_BQ, CODE_DIM), jnp.float32),
        pltpu.SemaphoreType.DMA,
    ],
    compiler_params=pltpu.CompilerParams(use_tc_tiling_on_sc=False),
)
def _sc_gather_kernel(cb_hbm, gidx_hbm, out_hbm, idx_v, rows_v, sem):
    wid = lax.axis_index("s") * 2 + lax.axis_index("c")
    g = wid // _QUARTERS
    q = wid % _QUARTERS
    pltpu.sync_copy(gidx_hbm.at[g // 2, g % 2, pl.ds(q * _BQ, _BQ)], idx_v)
    pltpu.async_copy(cb_hbm.at[idx_v], rows_v, sem).wait()
    pltpu.sync_copy(
        rows_v,
        out_hbm.at[pl.ds(q * _BQ, _BQ),
                   pl.ds(g * CODE_DIM, CODE_DIM)])


@jax.jit
def kernel(z, codebook):
    grid = (_PAIRS,)
    out_shapes = (
        jax.ShapeDtypeStruct((BATCH, NUM_CODEBOOKS), jnp.int32),
        jax.ShapeDtypeStruct((_PAIRS, 2, BATCH), jnp.int32),
        jax.ShapeDtypeStruct((1, CODEBOOK_SIZE), jnp.float32),
        jax.ShapeDtypeStruct((1, 128), jnp.float32),
    )
    indices, gidx, _counts, scal = pl.pallas_call(
        _vq_tc_kernel,
        grid=grid,
        in_specs=[
            pl.BlockSpec((BATCH, 2 * CODE_DIM), lambda j: (0, j)),
            pl.BlockSpec((2, CODEBOOK_SIZE, CODE_DIM), lambda j: (j, 0, 0)),
        ],
        out_specs=(
            pl.BlockSpec((BATCH, NUM_CODEBOOKS), lambda j: (0, 0)),
            pl.BlockSpec((1, 2, BATCH), lambda j: (j, 0, 0)),
            pl.BlockSpec((1, CODEBOOK_SIZE), lambda j: (0, 0)),
            pl.BlockSpec((1, 128), lambda j: (0, 0)),
        ),
        out_shape=out_shapes,
        scratch_shapes=[pltpu.SMEM((1,), jnp.float32)],
    )(z, codebook)

    cb_flat = codebook.reshape(NUM_CODEBOOKS * CODEBOOK_SIZE, CODE_DIM)
    quantized = _sc_gather_kernel(cb_flat, gidx)

    commitment_loss = scal[0, 0]
    codebook_loss = jnp.zeros((), dtype=jnp.float32)
    entropy = scal[0, 1]
    perplexity = scal[0, 2]
    return (quantized, indices, commitment_loss, codebook_loss,
            entropy, perplexity)


# speedup vs baseline: 1.7827x; 1.7827x over previous
import jax
import jax.numpy as jnp
from jax.experimental import pallas as pl
from jax.experimental.pallas import tpu as pltpu

NUM_CODEBOOKS = 8
CODEBOOK_SIZE = 512
CODE_DIM = 64
BATCH = 1024
_ONE = 1


def _vq_kernel(z_ref, cb_ref, zq_ref, idx_ref, counts_ref, scal_ref,
               commit_smem):
    j = pl.program_id(0)
    z2 = z_ref[...]

    @pl.when(j == 0)
    def _init():
        counts_ref[...] = jnp.zeros_like(counts_ref)
        commit_smem[0] = 0.0

    kf = jax.lax.broadcasted_iota(
        jnp.int32, (BATCH, CODEBOOK_SIZE), 1).astype(jnp.float32)
    lane_g = jax.lax.broadcasted_iota(jnp.int32, (BATCH, NUM_CODEBOOKS), 1)

    zq_halves = []
    cnt_total = None
    csum = jnp.sum(z2 * z2)
    for h in range(8):
        zg = z2[:, h * CODE_DIM:(h + 1) * CODE_DIM]
        c = cb_ref[h]
        c_sq = jnp.sum(c * c, axis=1, keepdims=True)
        cross = jax.lax.dot_general(
            zg, c, (((1,), (1,)), ((), ())),
            preferred_element_type=jnp.float32)
        d = jnp.transpose(c_sq) - 2.0 * cross
        dmin = jnp.min(d, axis=1, keepdims=True)
        idx_f = jnp.min(jnp.where(d <= dmin, kf, float(CODEBOOK_SIZE)),
                        axis=1, keepdims=True)
        one_hot = (kf == idx_f).astype(jnp.float32)
        zq_halves.append(jax.lax.dot_general(
            one_hot, c, (((1,), (0,)), ((), ())),
            preferred_element_type=jnp.float32))

        cnt = jnp.sum(one_hot, axis=0, keepdims=True)
        cnt_total = cnt if cnt_total is None else cnt_total + cnt
        csum = csum + jnp.sum(dmin)
        idx_col = idx_f.astype(jnp.int32)
        idx_ref[...] = jnp.where(lane_g == h, idx_col, idx_ref[...])

    zq_ref[...] = jnp.concatenate(zq_halves, axis=1)
    counts_ref[...] += cnt_total
    commit_smem[0] += csum

    @pl.when(j == 0)
    def _finish():
        usage = counts_ref[...] / float(BATCH * NUM_CODEBOOKS)
        ent = -jnp.sum(usage * jnp.log(usage + 1e-10))
        commit = commit_smem[0] / float(BATCH * NUM_CODEBOOKS * CODE_DIM)
        lane = jax.lax.broadcasted_iota(jnp.int32, (1, 128), 1)
        out = jnp.where(lane == 0, commit,
                        jnp.where(lane == 1, ent, jnp.exp(ent)))
        scal_ref[...] = out


@jax.jit
def kernel(z, codebook):
    grid = (_ONE,)
    out_shapes = (
        jax.ShapeDtypeStruct((BATCH, NUM_CODEBOOKS * CODE_DIM), jnp.float32),
        jax.ShapeDtypeStruct((BATCH, NUM_CODEBOOKS), jnp.int32),
        jax.ShapeDtypeStruct((1, CODEBOOK_SIZE), jnp.float32),
        jax.ShapeDtypeStruct((1, 128), jnp.float32),
    )
    quantized, indices, _counts, scal = pl.pallas_call(
        _vq_kernel,
        grid=grid,
        in_specs=[
            pl.BlockSpec((BATCH, 8 * CODE_DIM), lambda j: (0, j)),
            pl.BlockSpec((8, CODEBOOK_SIZE, CODE_DIM), lambda j: (j, 0, 0)),
        ],
        out_specs=(
            pl.BlockSpec((BATCH, 8 * CODE_DIM), lambda j: (0, j)),
            pl.BlockSpec((BATCH, NUM_CODEBOOKS), lambda j: (0, 0)),
            pl.BlockSpec((1, CODEBOOK_SIZE), lambda j: (0, 0)),
            pl.BlockSpec((1, 128), lambda j: (0, 0)),
        ),
        out_shape=out_shapes,
        scratch_shapes=[pltpu.SMEM((1,), jnp.float32)],
    )(z, codebook)

    commitment_loss = scal[0, 0]
    codebook_loss = jnp.zeros((), dtype=jnp.float32)
    entropy = scal[0, 1]
    perplexity = scal[0, 2]
    return (quantized, indices, commitment_loss, codebook_loss,
            entropy, perplexity)


# speedup vs baseline: 1.7874x; 1.0026x over previous
import jax
import jax.numpy as jnp
from jax.experimental import pallas as pl
from jax.experimental.pallas import tpu as pltpu

NUM_CODEBOOKS = 8
CODEBOOK_SIZE = 512
CODE_DIM = 64
BATCH = 1024
_QUADS = NUM_CODEBOOKS // 4


def _vq_kernel(z_ref, cb_ref, zq_ref, idx_ref, counts_ref, scal_ref,
               commit_smem):
    j = pl.program_id(0)
    z2 = z_ref[...]

    @pl.when(j == 0)
    def _init():
        counts_ref[...] = jnp.zeros_like(counts_ref)
        commit_smem[0] = 0.0

    kf = jax.lax.broadcasted_iota(
        jnp.int32, (BATCH, CODEBOOK_SIZE), 1).astype(jnp.float32)
    lane_g = jax.lax.broadcasted_iota(jnp.int32, (BATCH, NUM_CODEBOOKS), 1)

    zq_halves = []
    cnt_total = None
    csum = jnp.sum(z2 * z2)
    for h in range(4):
        zg = z2[:, h * CODE_DIM:(h + 1) * CODE_DIM]
        c = cb_ref[h]
        c_sq = jnp.sum(c * c, axis=1, keepdims=True)
        cross = jax.lax.dot_general(
            zg, c, (((1,), (1,)), ((), ())),
            preferred_element_type=jnp.float32)
        d = jnp.transpose(c_sq) - 2.0 * cross
        dmin = jnp.min(d, axis=1, keepdims=True)
        idx_f = jnp.min(jnp.where(d <= dmin, kf, float(CODEBOOK_SIZE)),
                        axis=1, keepdims=True)
        one_hot = (kf == idx_f).astype(jnp.float32)
        zq_halves.append(jax.lax.dot_general(
            one_hot, c, (((1,), (0,)), ((), ())),
            preferred_element_type=jnp.float32))

        cnt = jnp.sum(one_hot, axis=0, keepdims=True)
        cnt_total = cnt if cnt_total is None else cnt_total + cnt
        csum = csum + jnp.sum(dmin)
        idx_col = idx_f.astype(jnp.int32)
        idx_ref[...] = jnp.where(lane_g == 4 * j + h, idx_col, idx_ref[...])

    zq_ref[...] = jnp.concatenate(zq_halves, axis=1)
    counts_ref[...] += cnt_total
    commit_smem[0] += csum

    @pl.when(j == _QUADS - 1)
    def _finish():
        usage = counts_ref[...] / float(BATCH * NUM_CODEBOOKS)
        ent = -jnp.sum(usage * jnp.log(usage + 1e-10))
        commit = commit_smem[0] / float(BATCH * NUM_CODEBOOKS * CODE_DIM)
        lane = jax.lax.broadcasted_iota(jnp.int32, (1, 128), 1)
        out = jnp.where(lane == 0, commit,
                        jnp.where(lane == 1, ent, jnp.exp(ent)))
        scal_ref[...] = out


@jax.jit
def kernel(z, codebook):
    grid = (_QUADS,)
    out_shapes = (
        jax.ShapeDtypeStruct((BATCH, NUM_CODEBOOKS * CODE_DIM), jnp.float32),
        jax.ShapeDtypeStruct((BATCH, NUM_CODEBOOKS), jnp.int32),
        jax.ShapeDtypeStruct((1, CODEBOOK_SIZE), jnp.float32),
        jax.ShapeDtypeStruct((1, 128), jnp.float32),
    )
    quantized, indices, _counts, scal = pl.pallas_call(
        _vq_kernel,
        grid=grid,
        in_specs=[
            pl.BlockSpec((BATCH, 4 * CODE_DIM), lambda j: (0, j)),
            pl.BlockSpec((4, CODEBOOK_SIZE, CODE_DIM), lambda j: (j, 0, 0)),
        ],
        out_specs=(
            pl.BlockSpec((BATCH, 4 * CODE_DIM), lambda j: (0, j)),
            pl.BlockSpec((BATCH, NUM_CODEBOOKS), lambda j: (0, 0)),
            pl.BlockSpec((1, CODEBOOK_SIZE), lambda j: (0, 0)),
            pl.BlockSpec((1, 128), lambda j: (0, 0)),
        ),
        out_shape=out_shapes,
        scratch_shapes=[pltpu.SMEM((1,), jnp.float32)],
    )(z, codebook)

    commitment_loss = scal[0, 0]
    codebook_loss = jnp.zeros((), dtype=jnp.float32)
    entropy = scal[0, 1]
    perplexity = scal[0, 2]
    return (quantized, indices, commitment_loss, codebook_loss,
            entropy, perplexity)


# speedup vs baseline: 1.8905x; 1.0577x over previous
import jax
import jax.numpy as jnp
from jax.experimental import pallas as pl
from jax.experimental.pallas import tpu as pltpu

NUM_CODEBOOKS = 8
CODEBOOK_SIZE = 512
CODE_DIM = 64
BATCH = 1024
_PAIRS = NUM_CODEBOOKS // 2


def _vq_kernel(z_ref, cb_ref, zq_ref, idx_ref, counts_ref, scal_ref,
               commit_smem):
    j = pl.program_id(0)
    z2 = z_ref[...]

    @pl.when(j == 0)
    def _init():
        counts_ref[...] = jnp.zeros_like(counts_ref)
        commit_smem[0] = 0.0

    kf = jax.lax.broadcasted_iota(
        jnp.int32, (BATCH, CODEBOOK_SIZE), 1).astype(jnp.float32)
    lane_g = jax.lax.broadcasted_iota(jnp.int32, (BATCH, NUM_CODEBOOKS), 1)

    zq_halves = []
    cnt_total = None
    csum = jnp.sum(z2 * z2)
    for h in range(2):
        zg = z2[:, h * CODE_DIM:(h + 1) * CODE_DIM]
        c = cb_ref[h]
        c_sq = jnp.sum(c * c, axis=1, keepdims=True)
        cross = jax.lax.dot_general(
            zg, c, (((1,), (1,)), ((), ())),
            preferred_element_type=jnp.float32)
        d = jnp.transpose(c_sq) - 2.0 * cross
        dmin = jnp.min(d, axis=1, keepdims=True)
        idx_f = jnp.min(jnp.where(d <= dmin, kf, float(CODEBOOK_SIZE)),
                        axis=1, keepdims=True)
        one_hot = (kf == idx_f).astype(jnp.float32)
        zq_halves.append(jax.lax.dot_general(
            one_hot, c, (((1,), (0,)), ((), ())),
            preferred_element_type=jnp.float32))

        cnt = jnp.sum(one_hot, axis=0, keepdims=True)
        cnt_total = cnt if cnt_total is None else cnt_total + cnt
        csum = csum + jnp.sum(dmin)
        idx_col = idx_f.astype(jnp.int32)
        idx_ref[...] = jnp.where(lane_g == 2 * j + h, idx_col, idx_ref[...])

    zq_ref[...] = jnp.concatenate(zq_halves, axis=1)
    counts_ref[...] += cnt_total
    commit_smem[0] += csum

    @pl.when(j == _PAIRS - 1)
    def _finish():
        usage = counts_ref[...] / float(BATCH * NUM_CODEBOOKS)
        ent = -jnp.sum(usage * jnp.log(usage + 1e-10))
        scal_ref[0] = commit_smem[0] / float(BATCH * NUM_CODEBOOKS * CODE_DIM)
        scal_ref[1] = ent
        scal_ref[2] = jnp.exp(ent)
        scal_ref[3] = 0.0


@jax.jit
def kernel(z, codebook):
    grid = (_PAIRS,)
    out_shapes = (
        jax.ShapeDtypeStruct((BATCH, NUM_CODEBOOKS * CODE_DIM), jnp.float32),
        jax.ShapeDtypeStruct((BATCH, NUM_CODEBOOKS), jnp.int32),
        jax.ShapeDtypeStruct((1, CODEBOOK_SIZE), jnp.float32),
        jax.ShapeDtypeStruct((4,), jnp.float32),
    )
    quantized, indices, _counts, scal = pl.pallas_call(
        _vq_kernel,
        grid=grid,
        in_specs=[
            pl.BlockSpec((BATCH, 2 * CODE_DIM), lambda j: (0, j)),
            pl.BlockSpec((2, CODEBOOK_SIZE, CODE_DIM), lambda j: (j, 0, 0)),
        ],
        out_specs=(
            pl.BlockSpec((BATCH, 2 * CODE_DIM), lambda j: (0, j)),
            pl.BlockSpec((BATCH, NUM_CODEBOOKS), lambda j: (0, 0)),
            pl.BlockSpec((1, CODEBOOK_SIZE), lambda j: (0, 0)),
            pl.BlockSpec(memory_space=pltpu.SMEM),
        ),
        out_shape=out_shapes,
        scratch_shapes=[pltpu.SMEM((1,), jnp.float32)],
    )(z, codebook)

    commitment_loss = scal[0]
    codebook_loss = scal[3]
    entropy = scal[1]
    perplexity = scal[2]
    return (quantized, indices, commitment_loss, codebook_loss,
            entropy, perplexity)
